# baseline (speedup 1.0000x reference)
import functools

import jax
import jax.numpy as jnp
from jax import lax
from jax.experimental import pallas as pl
from jax.experimental.pallas import tpu as pltpu
from jax.experimental.pallas import tpu_sc as plsc

F32 = jnp.float32
I32 = jnp.int32

D = 32
KREF = 4
KD = KREF * D
NW = 32
CH = 112
NRING = 6

BN = 512
BE = 256
BD = 512


def _pad_rows(x, n_pad):
    return jnp.pad(x, ((0, n_pad - x.shape[0]),) + ((0, 0),) * (x.ndim - 1))


def _p1_body(hp_ref, ht_ref, wv_ref, bv_ref, vtab_ref, htab_ref):
    h = hp_ref[...] + ht_ref[...]
    v4 = jnp.dot(h, wv_ref[...], preferred_element_type=F32) + bv_ref[...]
    gi = lax.broadcasted_iota(I32, (KD, KREF), 0) // D
    gj = lax.broadcasted_iota(I32, (KD, KREF), 1)
    g = (gi == gj).astype(F32)
    g2i = lax.broadcasted_iota(I32, (KREF, KD), 0)
    g2j = lax.broadcasted_iota(I32, (KREF, KD), 1) // D
    g2 = (g2i == g2j).astype(F32)
    nrm2 = jnp.dot(v4 * v4, g, preferred_element_type=F32)
    inv = 1.0 / (jnp.sqrt(nrm2) + 1e-8)
    vn = v4 * jnp.dot(inv, g2, preferred_element_type=F32)
    s = h
    for j in range(KREF):
        vj = vn[:, D * j:D * j + D]
        dot = jnp.sum(vj * s, axis=1, keepdims=True)
        s = s - 2.0 * vj * dot
    vtab_ref[...] = vn
    htab_ref[...] = jnp.concatenate(
        [h, s, jnp.zeros((h.shape[0], KD - 2 * D), F32)], axis=1)


def _p1_call(h_prev, h_t, Wv, bv, n_pad):
    nspec = pl.BlockSpec((BN, D), lambda i: (i, 0))
    tspec = pl.BlockSpec((BN, KD), lambda i: (i, 0))
    shp = jax.ShapeDtypeStruct((n_pad, KD), F32)
    return pl.pallas_call(
        _p1_body,
        grid=(n_pad // BN,),
        in_specs=[nspec, nspec,
                  pl.BlockSpec((D, KD), lambda i: (0, 0)),
                  pl.BlockSpec((1, KD), lambda i: (0, 0))],
        out_specs=[tspec, tspec],
        out_shape=[shp, shp],
    )(h_prev, h_t, Wv, bv.reshape(1, KD))


def _ring_gather(tasks, bufs, gsems, ssems):
    n = len(tasks)
    nbuf = len(bufs)
    gdep = nbuf // 2
    gh = [None] * n
    sh = [None] * n
    sdone = [False] * n

    def issue(j):
        tab, idxs, _ = tasks[j]
        k = j % nbuf
        gh[j] = pltpu.async_copy(tab.at[idxs], bufs[k], gsems[k])

    for j in range(min(gdep, n)):
        issue(j)
    for j in range(n):
        k = j % nbuf
        gh[j].wait()
        _, _, (out, off) = tasks[j]
        sh[j] = pltpu.async_copy(bufs[k], out.at[pl.ds(off, CH)], ssems[k])
        jn = j + gdep
        if jn < n:
            prev = jn - nbuf
            if prev >= 0:
                sh[prev].wait()
                sdone[prev] = True
            issue(jn)
    for j in range(n):
        if sh[j] is not None and not sdone[j]:
            sh[j].wait()


def _sc_gather_body(e_pad, row_h, col_h, vtab_h, htab_h,
                    av_h, ah_h, bv_h, bz_h,
                    ridx_v, cidx_v, b0, b1, b2, b3, b4, b5,
                    g0, g1, g2, g3, g4, g5, s0, s1, s2, s3, s4, s5):
    wid = lax.axis_index("s") * 2 + lax.axis_index("c")
    per_w = e_pad // NW
    base = wid * per_w
    pltpu.sync_copy(row_h.at[pl.ds(base, per_w)], ridx_v)
    pltpu.sync_copy(col_h.at[pl.ds(base, per_w)], cidx_v)
    tasks = []
    for c in range(per_w // CH):
        off = base + c * CH
        rwin = ridx_v.at[pl.ds(c * CH, CH)]
        cwin = cidx_v.at[pl.ds(c * CH, CH)]
        tasks += [
            (vtab_h, rwin, (av_h, off)),
            (htab_h, rwin, (ah_h, off)),
            (vtab_h, cwin, (bv_h, off)),
            (htab_h, cwin, (bz_h, off)),
        ]
    _ring_gather(tasks, [b0, b1, b2, b3, b4, b5],
                 [g0, g1, g2, g3, g4, g5], [s0, s1, s2, s3, s4, s5])


def _p2_call(row_pad, col_pad, vtab, htab, e_pad):
    mesh = plsc.VectorSubcoreMesh(core_axis_name="c", subcore_axis_name="s")
    shp = jax.ShapeDtypeStruct((e_pad, KD), F32)
    per_w = e_pad // NW
    k = functools.partial(
        pl.kernel,
        mesh=mesh,
        out_type=[shp, shp, shp, shp],
        scratch_types=[
            pltpu.VMEM((per_w,), I32),
            pltpu.VMEM((per_w,), I32),
        ] + [pltpu.VMEM((CH, KD), F32)] * NRING
          + [pltpu.SemaphoreType.DMA] * (2 * NRING),
    )(functools.partial(_sc_gather_body, e_pad))
    return k(row_pad, col_pad, vtab, htab)


def _p3_compute(av_ref, ah_ref, bv_ref, bz_ref, w1_ref, b1_ref,
                w2_ref, b2_ref, q_ref, m_ref):
    av_t = jnp.transpose(av_ref[...])
    bv_t = jnp.transpose(bv_ref[...])
    hr = ah_ref[:, :D]
    zc_t = jnp.transpose(bz_ref[:, D:2 * D])
    a = [av_t[D * j:D * j + D] for j in range(KREF)]
    b = [bv_t[D * j:D * j + D] for j in range(KREF)]
    be = av_ref.shape[0]
    ii = lax.broadcasted_iota(I32, (D, D, be), 0)
    jj = lax.broadcasted_iota(I32, (D, D, be), 1)
    eye = (ii == jj).astype(F32)
    M = eye - 2.0 * b[0][:, None, :] * b[0][None, :, :]
    for v in [b[1], b[2], b[3], a[3], a[2], a[1], a[0]]:
        w = jnp.sum(M * v[:, None, :], axis=0)
        M = M - 2.0 * v[:, None, :] * w[None, :, :]
    q2d = jnp.transpose(M.reshape(D * D, be))
    q_ref[0] = q2d.reshape(be, 8, KD)
    s = zc_t
    for v in [a[3], a[2], a[1], a[0]]:
        dot = jnp.sum(v * s, axis=0, keepdims=True)
        s = s - 2.0 * v * dot
    msg_in = jnp.concatenate([hr, jnp.transpose(s)], axis=1)
    hid = jnp.dot(msg_in, w1_ref[...], preferred_element_type=F32) + b1_ref[...]
    hid = jnp.maximum(hid, 0.0)
    msg = jnp.dot(hid, w2_ref[...], preferred_element_type=F32) + b2_ref[...]
    m_ref[...] = jnp.concatenate(
        [msg, jnp.zeros((be, KD - D), F32)], axis=1)


def _p3_body0(av_ref, ah_ref, bv_ref, bz_ref, w1_ref, b1_ref,
              w2_ref, b2_ref, q_ref, m_ref):
    _p3_compute(av_ref, ah_ref, bv_ref, bz_ref, w1_ref, b1_ref,
                w2_ref, b2_ref, q_ref, m_ref)


def _p3_body1(qin_ref, av_ref, ah_ref, bv_ref, bz_ref, w1_ref, b1_ref,
              w2_ref, b2_ref, q_ref, m_ref):
    del qin_ref
    _p3_compute(av_ref, ah_ref, bv_ref, bz_ref, w1_ref, b1_ref,
                w2_ref, b2_ref, q_ref, m_ref)


def _p3_call(t, qbuf, av, ah, bvr, bz, W1, b1, W2, b2, t_frames, e, e_pad):
    espec = pl.BlockSpec((BE, KD), lambda i: (i, 0))
    wspecs = [
        pl.BlockSpec((2 * D, D), lambda i: (0, 0)),
        pl.BlockSpec((1, D), lambda i: (0, 0)),
        pl.BlockSpec((D, D), lambda i: (0, 0)),
        pl.BlockSpec((1, D), lambda i: (0, 0)),
    ]
    out_specs = [
        pl.BlockSpec((1, BE, 8, KD), lambda i: (t, i, 0, 0)),
        pl.BlockSpec((BE, KD), lambda i: (i, 0)),
    ]
    out_shape = [
        jax.ShapeDtypeStruct((t_frames, e, 8, KD), F32),
        jax.ShapeDtypeStruct((e_pad, KD), F32),
    ]
    args = (av, ah, bvr, bz, W1, b1.reshape(1, D), W2, b2.reshape(1, D))
    if t == 0:
        return pl.pallas_call(
            _p3_body0,
            grid=(e_pad // BE,),
            in_specs=[espec, espec, espec, espec] + wspecs,
            out_specs=out_specs,
            out_shape=out_shape,
        )(*args)
    return pl.pallas_call(
        _p3_body1,
        grid=(e_pad // BE,),
        in_specs=[pl.BlockSpec(memory_space=pl.ANY),
                  espec, espec, espec, espec] + wspecs,
        out_specs=out_specs,
        out_shape=out_shape,
        input_output_aliases={0: 0},
    )(qbuf, *args)


NB4 = 3
G4 = 2


def _sc_scatter_body(e_pad, n_pad, zeros_h, row_h, msgs_h, agg_h,
                     i0, i1, i2, m0, m1, m2, li0, li1, li2,
                     lm0, lm1, lm2, shared):
    cid = lax.axis_index("c")
    sid = lax.axis_index("s")
    wid = sid * 2 + cid
    rpt = n_pad // 16
    pltpu.sync_copy(zeros_h.at[pl.ds(sid * rpt, rpt)],
                    shared.at[pl.ds(sid * rpt, rpt)])
    plsc.subcore_barrier()
    per_w = e_pad // NW
    base = wid * per_w
    nch = per_w // CH
    idx_bufs = [i0, i1, i2]
    msg_bufs = [m0, m1, m2]
    isems = [li0, li1, li2]
    msems = [lm0, lm1, lm2]
    ih = [None] * nch
    mh = [None] * nch

    def issue(c):
        k = c % NB4
        off = base + c * CH
        ih[c] = pltpu.async_copy(row_h.at[pl.ds(off, CH)], idx_bufs[k],
                                 isems[k])
        mh[c] = pltpu.async_copy(msgs_h.at[pl.ds(off, CH)], msg_bufs[k],
                                 msems[k])

    for c in range(min(G4, nch)):
        issue(c)
    for c in range(nch):
        k = c % NB4
        ih[c].wait()
        mh[c].wait()
        pltpu.sync_copy(msg_bufs[k], shared.at[idx_bufs[k]], add=True)
        if c + G4 < nch:
            issue(c + G4)
    plsc.subcore_barrier()
    pltpu.sync_copy(shared.at[pl.ds(sid * rpt, rpt)],
                    agg_h.at[cid, pl.ds(sid * rpt, rpt)])


def _p4_call(zeros_nd, row_pad, msgs, e_pad, n_pad):
    mesh = plsc.VectorSubcoreMesh(core_axis_name="c", subcore_axis_name="s")
    k = functools.partial(
        pl.kernel,
        mesh=mesh,
        out_type=jax.ShapeDtypeStruct((2, n_pad, KD), F32),
        scratch_types=[pltpu.VMEM((CH,), I32)] * NB4
                    + [pltpu.VMEM((CH, KD), F32)] * NB4
                    + [pltpu.SemaphoreType.DMA] * (2 * NB4)
                    + [pltpu.VMEM_SHARED((n_pad, KD), F32)],
    )(functools.partial(_sc_scatter_body, e_pad, n_pad))
    return k(zeros_nd, row_pad, msgs)


def _p5_body(agg_ref, vtab_ref, htab_ref, wx_ref, bx_ref, wh_ref, bh_ref,
             gb_ref, hout_ref, wout_ref):
    agg = agg_ref[0, :, :D] + agg_ref[1, :, :D]
    vn = vtab_ref[...]
    h = htab_ref[:, :D]
    xp = jnp.dot(agg, wx_ref[...], preferred_element_type=F32) + bx_ref[...]
    hp = jnp.dot(h, wh_ref[...], preferred_element_type=F32) + bh_ref[...]
    r = jax.nn.sigmoid(xp[:, :D] + hp[:, :D])
    z = jax.nn.sigmoid(xp[:, D:2 * D] + hp[:, D:2 * D])
    n = jnp.tanh(xp[:, 2 * D:] + r * hp[:, 2 * D:])
    h_new = (1.0 - z) * n + z * h
    mu = jnp.mean(h_new, axis=1, keepdims=True)
    cen = h_new - mu
    var = jnp.mean(cen * cen, axis=1, keepdims=True)
    gamma = gb_ref[0:1, :]
    beta = gb_ref[1:2, :]
    h_cur = gamma * cen * jax.lax.rsqrt(var + 1e-5) + beta
    hout_ref[...] = h_cur
    s = h_cur
    for j in range(KREF):
        vj = vn[:, D * j:D * j + D]
        dot = jnp.sum(vj * s, axis=1, keepdims=True)
        s = s - 2.0 * vj * dot
    wout_ref[...] = jnp.concatenate(
        [s, jnp.zeros((s.shape[0], KD - D), F32)], axis=1)


def _p5_call(agg2, vtab, htab, Wx, bx, Wh, bh, gamma_beta, n_pad):
    return pl.pallas_call(
        _p5_body,
        grid=(n_pad // BN,),
        in_specs=[
            pl.BlockSpec((2, BN, KD), lambda i: (0, i, 0)),
            pl.BlockSpec((BN, KD), lambda i: (i, 0)),
            pl.BlockSpec((BN, KD), lambda i: (i, 0)),
            pl.BlockSpec((D, 3 * D), lambda i: (0, 0)),
            pl.BlockSpec((1, 3 * D), lambda i: (0, 0)),
            pl.BlockSpec((D, 3 * D), lambda i: (0, 0)),
            pl.BlockSpec((1, 3 * D), lambda i: (0, 0)),
            pl.BlockSpec((2, D), lambda i: (0, 0)),
        ],
        out_specs=[pl.BlockSpec((BN, D), lambda i: (i, 0)),
                   pl.BlockSpec((BN, KD), lambda i: (i, 0))],
        out_shape=[jax.ShapeDtypeStruct((n_pad, D), F32),
                   jax.ShapeDtypeStruct((n_pad, KD), F32)],
    )(agg2, vtab, htab, Wx, bx.reshape(1, 3 * D), Wh, bh.reshape(1, 3 * D),
      gamma_beta)


def _sc_gatherw_body(e_pad, row_h, col_h, wtab_h, wr_h, wc_h,
                     ridx_v, cidx_v, b0, b1, b2, b3, b4, b5,
                     g0, g1, g2, g3, g4, g5, s0, s1, s2, s3, s4, s5):
    wid = lax.axis_index("s") * 2 + lax.axis_index("c")
    per_w = e_pad // NW
    base = wid * per_w
    pltpu.sync_copy(row_h.at[pl.ds(base, per_w)], ridx_v)
    pltpu.sync_copy(col_h.at[pl.ds(base, per_w)], cidx_v)
    tasks = []
    for c in range(per_w // CH):
        off = base + c * CH
        tasks += [
            (wtab_h, ridx_v.at[pl.ds(c * CH, CH)], (wr_h, off)),
            (wtab_h, cidx_v.at[pl.ds(c * CH, CH)], (wc_h, off)),
        ]
    _ring_gather(tasks, [b0, b1, b2, b3, b4, b5],
                 [g0, g1, g2, g3, g4, g5], [s0, s1, s2, s3, s4, s5])


def _p6a_call(row_pad, col_pad, wtab, e_pad):
    mesh = plsc.VectorSubcoreMesh(core_axis_name="c", subcore_axis_name="s")
    shp = jax.ShapeDtypeStruct((e_pad, KD), F32)
    per_w = e_pad // NW
    k = functools.partial(
        pl.kernel,
        mesh=mesh,
        out_type=[shp, shp],
        scratch_types=[
            pltpu.VMEM((per_w,), I32),
            pltpu.VMEM((per_w,), I32),
        ] + [pltpu.VMEM((CH, KD), F32)] * NRING
          + [pltpu.SemaphoreType.DMA] * (2 * NRING),
    )(functools.partial(_sc_gatherw_body, e_pad))
    return k(row_pad, col_pad, wtab)


def _p6b_body(wr_ref, wc_ref, d_ref):
    dif = jnp.transpose(wr_ref[:, :D] - wc_ref[:, :D])
    d_ref[...] = jnp.sum(dif * dif, axis=0)[None, None, :]


def _p6b_call(wr, wc, e_pad):
    g = e_pad // BD
    out = pl.pallas_call(
        _p6b_body,
        grid=(g,),
        in_specs=[pl.BlockSpec((BD, KD), lambda i: (i, 0))] * 2,
        out_specs=pl.BlockSpec((1, 1, BD), lambda i: (i, 0, 0)),
        out_shape=jax.ShapeDtypeStruct((g, 1, BD), F32),
    )(wr, wc)
    return out.reshape(e_pad)


def kernel(h_sequence, edge_index_sequence, Wv, bv, W1, b1, W2, b2,
           Wx, bx, Wh, bh, gamma, beta):
    t_frames, n, d = h_sequence.shape
    e = edge_index_sequence.shape[2]
    n_pad = -(-n // BN) * BN
    e_pad = -(-e // (NW * CH)) * (NW * CH)

    zeros_nd = jnp.zeros((n_pad, KD), F32)
    gamma_beta = jnp.stack([gamma, beta])
    h_pad = [_pad_rows(h_sequence[t], n_pad) for t in range(t_frames)]

    h_prev = h_pad[0]
    qbuf = None
    all_h, all_d = [], []
    for t in range(t_frames):
        ei = edge_index_sequence[t]
        pad_idx = jnp.full((e_pad - e,), n, I32)
        row_pad = jnp.concatenate([ei[0], pad_idx])
        col_pad = jnp.concatenate([ei[1], pad_idx])

        vtab, htab = _p1_call(h_prev, h_pad[t], Wv, bv, n_pad)
        av, ah, bvr, bz = _p2_call(row_pad, col_pad, vtab, htab, e_pad)
        qbuf, msgs = _p3_call(t, qbuf, av, ah, bvr, bz, W1, b1, W2, b2,
                              t_frames, e, e_pad)
        agg2 = _p4_call(zeros_nd, row_pad, msgs, e_pad, n_pad)
        h_cur, wtab = _p5_call(agg2, vtab, htab, Wx, bx, Wh, bh,
                               gamma_beta, n_pad)
        wr, wc = _p6a_call(row_pad, col_pad, wtab, e_pad)
        d_t = _p6b_call(wr, wc, e_pad)

        all_h.append(h_cur[:n])
        all_d.append(d_t[:e])
        h_prev = _pad_rows(h_cur[:n], n_pad)

    return (all_h[-1], jnp.stack(all_h), jnp.stack(all_d),
            qbuf.reshape(t_frames, e, D, D))


# speedup vs baseline: 1.0166x; 1.0166x over previous
import functools

import jax
import jax.numpy as jnp
from jax import lax
from jax.experimental import pallas as pl
from jax.experimental.pallas import tpu as pltpu
from jax.experimental.pallas import tpu_sc as plsc

F32 = jnp.float32
I32 = jnp.int32

D = 32
KREF = 4
KD = KREF * D
NW = 32
CH = 112
NRING = 6

BN = 512
BE = 256
BD = 512


def _pad_rows(x, n_pad):
    return jnp.pad(x, ((0, n_pad - x.shape[0]),) + ((0, 0),) * (x.ndim - 1))


def _p1_core(h, wv_ref, bv_ref):
    v4 = jnp.dot(h, wv_ref[...], preferred_element_type=F32) + bv_ref[...]
    gi = lax.broadcasted_iota(I32, (KD, KREF), 0) // D
    gj = lax.broadcasted_iota(I32, (KD, KREF), 1)
    g = (gi == gj).astype(F32)
    g2i = lax.broadcasted_iota(I32, (KREF, KD), 0)
    g2j = lax.broadcasted_iota(I32, (KREF, KD), 1) // D
    g2 = (g2i == g2j).astype(F32)
    nrm2 = jnp.dot(v4 * v4, g, preferred_element_type=F32)
    inv = 1.0 / (jnp.sqrt(nrm2) + 1e-8)
    vn = v4 * jnp.dot(inv, g2, preferred_element_type=F32)
    s = h
    for j in range(KREF):
        vj = vn[:, D * j:D * j + D]
        dot = jnp.sum(vj * s, axis=1, keepdims=True)
        s = s - 2.0 * vj * dot
    return vn, jnp.concatenate(
        [h, s, jnp.zeros((h.shape[0], KD - 2 * D), F32)], axis=1)


def _p1_body(hp_ref, ht_ref, wv_ref, bv_ref, vtab_ref, htab_ref):
    vn, ht = _p1_core(hp_ref[...] + ht_ref[...], wv_ref, bv_ref)
    vtab_ref[...] = vn
    htab_ref[...] = ht


def _p1_call(h_prev, h_t, Wv, bv, n_pad):
    nspec = pl.BlockSpec((BN, D), lambda i: (i, 0))
    tspec = pl.BlockSpec((BN, KD), lambda i: (i, 0))
    shp = jax.ShapeDtypeStruct((n_pad, KD), F32)
    return pl.pallas_call(
        _p1_body,
        grid=(n_pad // BN,),
        in_specs=[nspec, nspec,
                  pl.BlockSpec((D, KD), lambda i: (0, 0)),
                  pl.BlockSpec((1, KD), lambda i: (0, 0))],
        out_specs=[tspec, tspec],
        out_shape=[shp, shp],
    )(h_prev, h_t, Wv, bv.reshape(1, KD))


def _ring_gather(tasks, bufs, gsems, ssems):
    n = len(tasks)
    nbuf = len(bufs)
    gdep = nbuf // 2
    gh = [None] * n
    sh = [None] * n
    sdone = [False] * n

    def issue(j):
        tab, idxs, _ = tasks[j]
        k = j % nbuf
        gh[j] = pltpu.async_copy(tab.at[idxs], bufs[k], gsems[k])

    for j in range(min(gdep, n)):
        issue(j)
    for j in range(n):
        k = j % nbuf
        gh[j].wait()
        _, _, (out, off) = tasks[j]
        sh[j] = pltpu.async_copy(bufs[k], out.at[pl.ds(off, CH)], ssems[k])
        jn = j + gdep
        if jn < n:
            prev = jn - nbuf
            if prev >= 0:
                sh[prev].wait()
                sdone[prev] = True
            issue(jn)
    for j in range(n):
        if sh[j] is not None and not sdone[j]:
            sh[j].wait()


def _sc_gather_body(e_pad, row_h, col_h, vtab_h, htab_h,
                    av_h, ah_h, bv_h, bz_h,
                    ridx_v, cidx_v, b0, b1, b2, b3, b4, b5,
                    g0, g1, g2, g3, g4, g5, s0, s1, s2, s3, s4, s5):
    wid = lax.axis_index("s") * 2 + lax.axis_index("c")
    per_w = e_pad // NW
    base = wid * per_w
    pltpu.sync_copy(row_h.at[pl.ds(base, per_w)], ridx_v)
    pltpu.sync_copy(col_h.at[pl.ds(base, per_w)], cidx_v)
    tasks = []
    for c in range(per_w // CH):
        off = base + c * CH
        rwin = ridx_v.at[pl.ds(c * CH, CH)]
        cwin = cidx_v.at[pl.ds(c * CH, CH)]
        tasks += [
            (vtab_h, rwin, (av_h, off)),
            (htab_h, rwin, (ah_h, off)),
            (vtab_h, cwin, (bv_h, off)),
            (htab_h, cwin, (bz_h, off)),
        ]
    _ring_gather(tasks, [b0, b1, b2, b3, b4, b5],
                 [g0, g1, g2, g3, g4, g5], [s0, s1, s2, s3, s4, s5])


def _p2_call(row_pad, col_pad, vtab, htab, e_pad):
    mesh = plsc.VectorSubcoreMesh(core_axis_name="c", subcore_axis_name="s")
    shp = jax.ShapeDtypeStruct((e_pad, KD), F32)
    per_w = e_pad // NW
    k = functools.partial(
        pl.kernel,
        mesh=mesh,
        out_type=[shp, shp, shp, shp],
        scratch_types=[
            pltpu.VMEM((per_w,), I32),
            pltpu.VMEM((per_w,), I32),
        ] + [pltpu.VMEM((CH, KD), F32)] * NRING
          + [pltpu.SemaphoreType.DMA] * (2 * NRING),
    )(functools.partial(_sc_gather_body, e_pad))
    return k(row_pad, col_pad, vtab, htab)


def _p3_compute(av_ref, ah_ref, bv_ref, bz_ref, w1_ref, b1_ref,
                w2_ref, b2_ref, q_ref, m_ref):
    av_t = jnp.transpose(av_ref[...])
    bv_t = jnp.transpose(bv_ref[...])
    hr = ah_ref[:, :D]
    zc_t = jnp.transpose(bz_ref[:, D:2 * D])
    a = [av_t[D * j:D * j + D] for j in range(KREF)]
    b = [bv_t[D * j:D * j + D] for j in range(KREF)]
    be = av_ref.shape[0]
    ii = lax.broadcasted_iota(I32, (D, D, be), 0)
    jj = lax.broadcasted_iota(I32, (D, D, be), 1)
    eye = (ii == jj).astype(F32)
    M = eye - 2.0 * b[0][:, None, :] * b[0][None, :, :]
    for v in [b[1], b[2], b[3], a[3], a[2], a[1], a[0]]:
        w = jnp.sum(M * v[:, None, :], axis=0)
        M = M - 2.0 * v[:, None, :] * w[None, :, :]
    q2d = jnp.transpose(M.reshape(D * D, be))
    q_ref[0] = q2d.reshape(be, 8, KD)
    s = zc_t
    for v in [a[3], a[2], a[1], a[0]]:
        dot = jnp.sum(v * s, axis=0, keepdims=True)
        s = s - 2.0 * v * dot
    msg_in = jnp.concatenate([hr, jnp.transpose(s)], axis=1)
    hid = jnp.dot(msg_in, w1_ref[...], preferred_element_type=F32) + b1_ref[...]
    hid = jnp.maximum(hid, 0.0)
    msg = jnp.dot(hid, w2_ref[...], preferred_element_type=F32) + b2_ref[...]
    m_ref[...] = jnp.concatenate(
        [msg, jnp.zeros((be, KD - D), F32)], axis=1)


def _p3_body0(av_ref, ah_ref, bv_ref, bz_ref, w1_ref, b1_ref,
              w2_ref, b2_ref, q_ref, m_ref):
    _p3_compute(av_ref, ah_ref, bv_ref, bz_ref, w1_ref, b1_ref,
                w2_ref, b2_ref, q_ref, m_ref)


def _p3_body1(qin_ref, av_ref, ah_ref, bv_ref, bz_ref, w1_ref, b1_ref,
              w2_ref, b2_ref, q_ref, m_ref):
    del qin_ref
    _p3_compute(av_ref, ah_ref, bv_ref, bz_ref, w1_ref, b1_ref,
                w2_ref, b2_ref, q_ref, m_ref)


def _p3_call(t, qbuf, av, ah, bvr, bz, W1, b1, W2, b2, t_frames, e, e_pad):
    espec = pl.BlockSpec((BE, KD), lambda i: (i, 0))
    wspecs = [
        pl.BlockSpec((2 * D, D), lambda i: (0, 0)),
        pl.BlockSpec((1, D), lambda i: (0, 0)),
        pl.BlockSpec((D, D), lambda i: (0, 0)),
        pl.BlockSpec((1, D), lambda i: (0, 0)),
    ]
    out_specs = [
        pl.BlockSpec((1, BE, 8, KD), lambda i: (t, i, 0, 0)),
        pl.BlockSpec((BE, KD), lambda i: (i, 0)),
    ]
    out_shape = [
        jax.ShapeDtypeStruct((t_frames, e, 8, KD), F32),
        jax.ShapeDtypeStruct((e_pad, KD), F32),
    ]
    args = (av, ah, bvr, bz, W1, b1.reshape(1, D), W2, b2.reshape(1, D))
    if t == 0:
        return pl.pallas_call(
            _p3_body0,
            grid=(e_pad // BE,),
            in_specs=[espec, espec, espec, espec] + wspecs,
            out_specs=out_specs,
            out_shape=out_shape,
        )(*args)
    return pl.pallas_call(
        _p3_body1,
        grid=(e_pad // BE,),
        in_specs=[pl.BlockSpec(memory_space=pl.ANY),
                  espec, espec, espec, espec] + wspecs,
        out_specs=out_specs,
        out_shape=out_shape,
        input_output_aliases={0: 0},
    )(qbuf, *args)


NB4 = 3
G4 = 2


def _sc_scatter_body(e_pad, n_pad, zeros_h, row_h, msgs_h, agg_h,
                     i0, i1, i2, m0, m1, m2, li0, li1, li2,
                     lm0, lm1, lm2, shared):
    cid = lax.axis_index("c")
    sid = lax.axis_index("s")
    wid = sid * 2 + cid
    rpt = n_pad // 16
    pltpu.sync_copy(zeros_h.at[pl.ds(sid * rpt, rpt)],
                    shared.at[pl.ds(sid * rpt, rpt)])
    plsc.subcore_barrier()
    per_w = e_pad // NW
    base = wid * per_w
    nch = per_w // CH
    idx_bufs = [i0, i1, i2]
    msg_bufs = [m0, m1, m2]
    isems = [li0, li1, li2]
    msems = [lm0, lm1, lm2]
    ih = [None] * nch
    mh = [None] * nch

    def issue(c):
        k = c % NB4
        off = base + c * CH
        ih[c] = pltpu.async_copy(row_h.at[pl.ds(off, CH)], idx_bufs[k],
                                 isems[k])
        mh[c] = pltpu.async_copy(msgs_h.at[pl.ds(off, CH)], msg_bufs[k],
                                 msems[k])

    for c in range(min(G4, nch)):
        issue(c)
    for c in range(nch):
        k = c % NB4
        ih[c].wait()
        mh[c].wait()
        pltpu.sync_copy(msg_bufs[k], shared.at[idx_bufs[k]], add=True)
        if c + G4 < nch:
            issue(c + G4)
    plsc.subcore_barrier()
    pltpu.sync_copy(shared.at[pl.ds(sid * rpt, rpt)],
                    agg_h.at[cid, pl.ds(sid * rpt, rpt)])


def _p4_call(zeros_nd, row_pad, msgs, e_pad, n_pad):
    mesh = plsc.VectorSubcoreMesh(core_axis_name="c", subcore_axis_name="s")
    k = functools.partial(
        pl.kernel,
        mesh=mesh,
        out_type=jax.ShapeDtypeStruct((2, n_pad, KD), F32),
        scratch_types=[pltpu.VMEM((CH,), I32)] * NB4
                    + [pltpu.VMEM((CH, KD), F32)] * NB4
                    + [pltpu.SemaphoreType.DMA] * (2 * NB4)
                    + [pltpu.VMEM_SHARED((n_pad, KD), F32)],
    )(functools.partial(_sc_scatter_body, e_pad, n_pad))
    return k(zeros_nd, row_pad, msgs)


def _p5_core(agg_ref, vtab_ref, htab_ref, wx_ref, bx_ref, wh_ref, bh_ref,
             gb_ref):
    agg = agg_ref[0, :, :D] + agg_ref[1, :, :D]
    vn = vtab_ref[...]
    h = htab_ref[:, :D]
    xp = jnp.dot(agg, wx_ref[...], preferred_element_type=F32) + bx_ref[...]
    hp = jnp.dot(h, wh_ref[...], preferred_element_type=F32) + bh_ref[...]
    r = jax.nn.sigmoid(xp[:, :D] + hp[:, :D])
    z = jax.nn.sigmoid(xp[:, D:2 * D] + hp[:, D:2 * D])
    n = jnp.tanh(xp[:, 2 * D:] + r * hp[:, 2 * D:])
    h_new = (1.0 - z) * n + z * h
    mu = jnp.mean(h_new, axis=1, keepdims=True)
    cen = h_new - mu
    var = jnp.mean(cen * cen, axis=1, keepdims=True)
    gamma = gb_ref[0:1, :]
    beta = gb_ref[1:2, :]
    h_cur = gamma * cen * jax.lax.rsqrt(var + 1e-5) + beta
    s = h_cur
    for j in range(KREF):
        vj = vn[:, D * j:D * j + D]
        dot = jnp.sum(vj * s, axis=1, keepdims=True)
        s = s - 2.0 * vj * dot
    return h_cur, jnp.concatenate(
        [s, jnp.zeros((s.shape[0], KD - D), F32)], axis=1)


def _p5_body(agg_ref, vtab_ref, htab_ref, wx_ref, bx_ref, wh_ref, bh_ref,
             gb_ref, hout_ref, wout_ref):
    h_cur, w = _p5_core(agg_ref, vtab_ref, htab_ref, wx_ref, bx_ref,
                        wh_ref, bh_ref, gb_ref)
    hout_ref[...] = h_cur
    wout_ref[...] = w


def _p5f_body(agg_ref, vtab_ref, htab_ref, hnext_ref, wx_ref, bx_ref,
              wh_ref, bh_ref, gb_ref, wv_ref, bv_ref,
              hout_ref, wout_ref, vtab2_ref, htab2_ref):
    h_cur, w = _p5_core(agg_ref, vtab_ref, htab_ref, wx_ref, bx_ref,
                        wh_ref, bh_ref, gb_ref)
    hout_ref[...] = h_cur
    wout_ref[...] = w
    vn2, ht2 = _p1_core(h_cur + hnext_ref[...], wv_ref, bv_ref)
    vtab2_ref[...] = vn2
    htab2_ref[...] = ht2


def _p5_call(agg2, vtab, htab, h_next, Wx, bx, Wh, bh, gamma_beta,
             Wv, bv, n_pad):
    nspec = pl.BlockSpec((BN, D), lambda i: (i, 0))
    tspec = pl.BlockSpec((BN, KD), lambda i: (i, 0))
    in_specs = [
        pl.BlockSpec((2, BN, KD), lambda i: (0, i, 0)),
        tspec, tspec,
        pl.BlockSpec((D, 3 * D), lambda i: (0, 0)),
        pl.BlockSpec((1, 3 * D), lambda i: (0, 0)),
        pl.BlockSpec((D, 3 * D), lambda i: (0, 0)),
        pl.BlockSpec((1, 3 * D), lambda i: (0, 0)),
        pl.BlockSpec((2, D), lambda i: (0, 0)),
    ]
    args = (agg2, vtab, htab, Wx, bx.reshape(1, 3 * D), Wh,
            bh.reshape(1, 3 * D), gamma_beta)
    hshp = jax.ShapeDtypeStruct((n_pad, D), F32)
    tshp = jax.ShapeDtypeStruct((n_pad, KD), F32)
    if h_next is None:
        return pl.pallas_call(
            _p5_body,
            grid=(n_pad // BN,),
            in_specs=in_specs,
            out_specs=[nspec, tspec],
            out_shape=[hshp, tshp],
        )(*args)
    return pl.pallas_call(
        _p5f_body,
        grid=(n_pad // BN,),
        in_specs=in_specs[:3] + [nspec] + in_specs[3:]
        + [pl.BlockSpec((D, KD), lambda i: (0, 0)),
           pl.BlockSpec((1, KD), lambda i: (0, 0))],
        out_specs=[nspec, tspec, tspec, tspec],
        out_shape=[hshp, tshp, tshp, tshp],
    )(agg2, vtab, htab, h_next, Wx, bx.reshape(1, 3 * D), Wh,
      bh.reshape(1, 3 * D), gamma_beta, Wv, bv.reshape(1, KD))


def _sc_gatherw_body(e_pad, row_h, col_h, wtab_h, wr_h, wc_h,
                     ridx_v, cidx_v, b0, b1, b2, b3, b4, b5,
                     g0, g1, g2, g3, g4, g5, s0, s1, s2, s3, s4, s5):
    wid = lax.axis_index("s") * 2 + lax.axis_index("c")
    per_w = e_pad // NW
    base = wid * per_w
    pltpu.sync_copy(row_h.at[pl.ds(base, per_w)], ridx_v)
    pltpu.sync_copy(col_h.at[pl.ds(base, per_w)], cidx_v)
    tasks = []
    for c in range(per_w // CH):
        off = base + c * CH
        tasks += [
            (wtab_h, ridx_v.at[pl.ds(c * CH, CH)], (wr_h, off)),
            (wtab_h, cidx_v.at[pl.ds(c * CH, CH)], (wc_h, off)),
        ]
    _ring_gather(tasks, [b0, b1, b2, b3, b4, b5],
                 [g0, g1, g2, g3, g4, g5], [s0, s1, s2, s3, s4, s5])


def _p6a_call(row_pad, col_pad, wtab, e_pad):
    mesh = plsc.VectorSubcoreMesh(core_axis_name="c", subcore_axis_name="s")
    shp = jax.ShapeDtypeStruct((e_pad, KD), F32)
    per_w = e_pad // NW
    k = functools.partial(
        pl.kernel,
        mesh=mesh,
        out_type=[shp, shp],
        scratch_types=[
            pltpu.VMEM((per_w,), I32),
            pltpu.VMEM((per_w,), I32),
        ] + [pltpu.VMEM((CH, KD), F32)] * NRING
          + [pltpu.SemaphoreType.DMA] * (2 * NRING),
    )(functools.partial(_sc_gatherw_body, e_pad))
    return k(row_pad, col_pad, wtab)


def _p6b_body(wr_ref, wc_ref, d_ref):
    dif = jnp.transpose(wr_ref[:, :D] - wc_ref[:, :D])
    d_ref[...] = jnp.sum(dif * dif, axis=0)[None, None, :]


def _p6b_call(wr, wc, e_pad):
    g = e_pad // BD
    out = pl.pallas_call(
        _p6b_body,
        grid=(g,),
        in_specs=[pl.BlockSpec((BD, KD), lambda i: (i, 0))] * 2,
        out_specs=pl.BlockSpec((1, 1, BD), lambda i: (i, 0, 0)),
        out_shape=jax.ShapeDtypeStruct((g, 1, BD), F32),
    )(wr, wc)
    return out.reshape(e_pad)


def kernel(h_sequence, edge_index_sequence, Wv, bv, W1, b1, W2, b2,
           Wx, bx, Wh, bh, gamma, beta):
    t_frames, n, d = h_sequence.shape
    e = edge_index_sequence.shape[2]
    n_pad = -(-n // BN) * BN
    e_pad = -(-e // (NW * CH)) * (NW * CH)

    zeros_nd = jnp.zeros((n_pad, KD), F32)
    gamma_beta = jnp.stack([gamma, beta])
    h_pad = [_pad_rows(h_sequence[t], n_pad) for t in range(t_frames)]
    ei_pad = jnp.pad(edge_index_sequence, ((0, 0), (0, 0), (0, e_pad - e)),
                     constant_values=n)

    qbuf = None
    vtab, htab = _p1_call(h_pad[0], h_pad[0], Wv, bv, n_pad)
    all_h, all_d = [], []
    for t in range(t_frames):
        row_pad = ei_pad[t, 0]
        col_pad = ei_pad[t, 1]

        av, ah, bvr, bz = _p2_call(row_pad, col_pad, vtab, htab, e_pad)
        qbuf, msgs = _p3_call(t, qbuf, av, ah, bvr, bz, W1, b1, W2, b2,
                              t_frames, e, e_pad)
        agg2 = _p4_call(zeros_nd, row_pad, msgs, e_pad, n_pad)
        h_next = h_pad[t + 1] if t + 1 < t_frames else None
        out5 = _p5_call(agg2, vtab, htab, h_next, Wx, bx, Wh, bh,
                        gamma_beta, Wv, bv, n_pad)
        if h_next is None:
            h_cur, wtab = out5
        else:
            h_cur, wtab, vtab, htab = out5
        wr, wc = _p6a_call(row_pad, col_pad, wtab, e_pad)
        d_t = _p6b_call(wr, wc, e_pad)

        all_h.append(h_cur[:n])
        all_d.append(d_t[:e])

    return (all_h[-1], jnp.stack(all_h), jnp.stack(all_d),
            qbuf.reshape(t_frames, e, D, D))
